# baseline (speedup 1.0000x reference)
import functools

import jax
import jax.numpy as jnp
from jax import lax
from jax.experimental import pallas as pl
from jax.experimental.pallas import tpu as pltpu
from jax.experimental.pallas import tpu_sc as plsc

D = 128
NBUF = 8
PRE = 5


@functools.partial(jax.jit, static_argnames=("s", "nc", "ns"))
def _gather_sc(ids, table, s, nc, ns):
    nw = nc * ns
    n_seq = ids.shape[0]
    per_w = n_seq // nw
    mesh = plsc.VectorSubcoreMesh(core_axis_name="c", subcore_axis_name="s")

    @functools.partial(
        pl.kernel,
        mesh=mesh,
        out_type=jax.ShapeDtypeStruct((n_seq, s, D), jnp.float32),
        scratch_types=(
            [pltpu.VMEM((per_w, s), jnp.int32),
             pltpu.VMEM((NBUF, s, D), jnp.float32)]
            + [pltpu.SemaphoreType.DMA] * (2 * NBUF)
        ),
    )
    def k(ids_hbm, table_hbm, out_hbm, idx_v, rows_v, *sems):
        gsems, ssems = sems[:NBUF], sems[NBUF:]
        wid = lax.axis_index("s") * nc + lax.axis_index("c")
        base = wid * per_w
        pltpu.sync_copy(ids_hbm.at[pl.ds(base, per_w)], idx_v)

        def start_gather(j, b):
            pltpu.async_copy(table_hbm.at[idx_v.at[j]], rows_v.at[b], gsems[b])

        def wait_gather(b):
            pltpu.make_async_copy(
                table_hbm.at[idx_v.at[0]], rows_v.at[b], gsems[b]).wait()

        def start_store(j, b):
            pltpu.async_copy(rows_v.at[b], out_hbm.at[base + j], ssems[b])

        def wait_store(b):
            pltpu.make_async_copy(
                rows_v.at[b], out_hbm.at[base], ssems[b]).wait()

        for j in range(PRE):
            start_gather(j, j % NBUF)

        def outer(i, carry):
            g = i * NBUF
            for b in range(NBUF):
                j = g + b
                wait_gather(b)
                start_store(j, b)
                jn = j + PRE
                bn = (b + PRE) % NBUF

                @pl.when(jn < per_w)
                def _():
                    @pl.when(jn >= NBUF)
                    def _():
                        wait_store(bn)
                    start_gather(jn, bn)

            return carry

        lax.fori_loop(0, per_w // NBUF, outer, 0)
        for b in range(NBUF):
            wait_store(b)

    return k(ids, table)


K = 4


def kernel(input_ids, table):
    b0, s = input_ids.shape
    info = plsc.get_sparse_core_info()
    nc, ns = info.num_cores, info.num_subcores
    ids = input_ids.astype(jnp.int32)
    step = b0 // K
    outs = [_gather_sc(ids[k * step:(k + 1) * step], table, s, nc, ns)
            for k in range(K)]
    return jnp.concatenate(outs, axis=0)


# speedup vs baseline: 1.0157x; 1.0157x over previous
import functools

import jax
import jax.numpy as jnp
from jax import lax
from jax.experimental import pallas as pl
from jax.experimental.pallas import tpu as pltpu
from jax.experimental.pallas import tpu_sc as plsc

D = 128
NBUF = 8
PRE = 5


@functools.partial(jax.jit, static_argnames=("s", "nc", "ns"))
def _gather_sc(ids, table, s, nc, ns):
    nw = nc * ns
    n_seq = ids.shape[0]
    per_w = n_seq // nw
    mesh = plsc.VectorSubcoreMesh(core_axis_name="c", subcore_axis_name="s")

    @functools.partial(
        pl.kernel,
        mesh=mesh,
        out_type=jax.ShapeDtypeStruct((n_seq, s, D), jnp.float32),
        scratch_types=(
            [pltpu.VMEM((per_w, s), jnp.int32),
             pltpu.VMEM((NBUF, s, D), jnp.float32)]
            + [pltpu.SemaphoreType.DMA] * (2 * NBUF)
        ),
    )
    def k(ids_hbm, table_hbm, out_hbm, idx_v, rows_v, *sems):
        gsems, ssems = sems[:NBUF], sems[NBUF:]
        wid = lax.axis_index("s") * nc + lax.axis_index("c")
        base = wid * per_w
        pltpu.sync_copy(ids_hbm.at[pl.ds(base, per_w)], idx_v)

        def start_gather(j, b):
            pltpu.async_copy(table_hbm.at[idx_v.at[j]], rows_v.at[b], gsems[b])

        def wait_gather(b):
            pltpu.make_async_copy(
                table_hbm.at[idx_v.at[0]], rows_v.at[b], gsems[b]).wait()

        def start_store(j, b):
            pltpu.async_copy(rows_v.at[b], out_hbm.at[base + j], ssems[b])

        def wait_store(b):
            pltpu.make_async_copy(
                rows_v.at[b], out_hbm.at[base], ssems[b]).wait()

        for j in range(PRE):
            start_gather(j, j % NBUF)

        def outer(i, carry):
            g = i * NBUF
            for b in range(NBUF):
                j = g + b
                wait_gather(b)
                start_store(j, b)
                jn = j + PRE
                bn = (b + PRE) % NBUF

                @pl.when(jn < per_w)
                def _():
                    @pl.when(jn >= NBUF)
                    def _():
                        wait_store(bn)
                    start_gather(jn, bn)

            return carry

        lax.fori_loop(0, per_w // NBUF, outer, 0)
        for b in range(NBUF):
            wait_store(b)

    return k(ids, table)


K = 4


def kernel(input_ids, table):
    b0, s = input_ids.shape
    info = plsc.get_sparse_core_info()
    nc, ns = info.num_cores, info.num_subcores
    ids = input_ids.astype(jnp.int32)
    step = b0 // K
    outs = [_gather_sc(ids[k * step:(k + 1) * step], table, s, nc, ns)
            for k in range(K)]
    acc = jnp.empty((b0, s, D), jnp.float32)
    for k in range(K):
        acc = lax.dynamic_update_slice(acc, outs[k], (k * step, 0, 0))
    return acc
